# baseline (speedup 1.0000x reference)
import functools

import jax
import jax.numpy as jnp
import numpy as np
from jax.experimental import pallas as pl
from jax.experimental.pallas import tpu as pltpu

B = 2
SEQ_LEN = 1536
PRED_LEN = 512
ENC_IN = 21
C_OUT = 21
D_MODEL = 1024
N_HEADS = 16
D_FF = 2048
E_LAYERS = 2
MARK = 4
BUCKET = 16
N_HASHES = 4
L = SEQ_LEN + PRED_LEN
D_HEAD = D_MODEL // N_HEADS
N_BUCKETS = L // BUCKET
BH = B * N_HEADS
RB = 512
NRB = (B * L) // RB
GQ = 128
NG = (N_HASHES * L) // GQ


def _pos_emb_np():
    pos = np.arange(L)[:, None].astype(np.float64)
    i = np.arange(0, D_MODEL, 2)[None, :].astype(np.float64)
    ang = pos / np.power(10000.0, i / D_MODEL)
    pe = np.zeros((L, D_MODEL), dtype=np.float32)
    pe[:, 0::2] = np.sin(ang)
    pe[:, 1::2] = np.cos(ang)
    return pe


def _ln(x):
    m = x.mean(-1, keepdims=True)
    v = ((x - m) ** 2).mean(-1, keepdims=True)
    return (x - m) / jnp.sqrt(v + 1e-5)


def _embed_body(xu_ref, w_ref, pos_ref, out_ref):
    out_ref[...] = (
        jnp.dot(xu_ref[...], w_ref[...], preferred_element_type=jnp.float32)
        + pos_ref[...]
    )


def _embed(xu, wcat, posb):
    return pl.pallas_call(
        _embed_body,
        grid=(NRB,),
        in_specs=[
            pl.BlockSpec((RB, 72), lambda r: (r, 0)),
            pl.BlockSpec((72, D_MODEL), lambda r: (0, 0)),
            pl.BlockSpec((RB, D_MODEL), lambda r: (r, 0)),
        ],
        out_specs=pl.BlockSpec((RB, D_MODEL), lambda r: (r, 0)),
        out_shape=jax.ShapeDtypeStruct((B * L, D_MODEL), jnp.float32),
    )(xu, wcat, posb)


def _qkv_body(x_ref, w_ref, v_ref):
    y = jnp.dot(x_ref[...], w_ref[...], preferred_element_type=jnp.float32)
    for h in range(N_HEADS):
        v_ref[h] = y[:, h * D_HEAD:(h + 1) * D_HEAD]


def _qkv(h, wv):
    out_spec = pl.BlockSpec(
        (N_HEADS, RB, D_HEAD), lambda r: (r // 4, r % 4, 0)
    )
    return pl.pallas_call(
        _qkv_body,
        grid=(NRB,),
        in_specs=[
            pl.BlockSpec((RB, D_MODEL), lambda r: (r, 0)),
            pl.BlockSpec((D_MODEL, D_MODEL), lambda r: (0, 0)),
        ],
        out_specs=out_spec,
        out_shape=jax.ShapeDtypeStruct((BH, L, D_HEAD), jnp.float32),
    )(h, wv)


def _bucket_body(qk_ref, r2_ref, out_ref):
    s = jnp.dot(qk_ref[0], r2_ref[...], preferred_element_type=jnp.float32)
    cols = []
    for h in range(N_HASHES):
        cols.append(jnp.argmax(s[:, h * N_BUCKETS:(h + 1) * N_BUCKETS], -1))
    out_ref[0] = jnp.stack(cols, axis=0).astype(jnp.int32)


def _buckets(qk_bh, r2):
    return pl.pallas_call(
        _bucket_body,
        grid=(BH,),
        in_specs=[
            pl.BlockSpec((1, L, D_HEAD), lambda b: (b, 0, 0)),
            pl.BlockSpec((D_HEAD, N_HASHES * N_BUCKETS), lambda b: (0, 0)),
        ],
        out_specs=pl.BlockSpec((1, N_HASHES, L), lambda b: (b, 0, 0)),
        out_shape=jax.ShapeDtypeStruct((BH, N_HASHES, L), jnp.int32),
    )(qk_bh, r2)


def _attn_body(qc_ref, qp_ref, vc_ref, vp_ref, t_ref, so_ref, lg_ref):
    g = pl.program_id(1)
    q = qc_ref[0]
    kc = q / (jnp.sqrt((q * q).sum(-1, keepdims=True)) + 1e-9)
    qp = qp_ref[0][GQ - BUCKET:]
    kp = qp / (jnp.sqrt((qp * qp).sum(-1, keepdims=True)) + 1e-9)
    kprev = jnp.concatenate([kp, kc[:GQ - BUCKET]], axis=0)
    k2 = jnp.concatenate([kc, kprev], axis=0)
    vc = vc_ref[0]
    vp = vp_ref[0][GQ - BUCKET:]
    v2 = jnp.concatenate(
        [vc, jnp.concatenate([vp, vc[:GQ - BUCKET]], axis=0)], axis=0
    )
    tcur = t_ref[0, pl.ds(g, 1), :][0]
    tprow = t_ref[0, pl.ds((g + NG - 1) % NG, 1), :][0]
    tprev = jnp.concatenate([tprow[GQ - BUCKET:], tcur[:GQ - BUCKET]])
    t2 = jnp.concatenate([tcur, tprev])

    d = jnp.dot(q, k2.T, preferred_element_type=jnp.float32) * (D_HEAD ** -0.5)
    ci = jax.lax.broadcasted_iota(jnp.int32, (GQ, 2 * GQ), 0) // BUCKET
    cj = (jax.lax.broadcasted_iota(jnp.int32, (GQ, 2 * GQ), 1) % GQ) // BUCKET
    d = jnp.where(tcur[:, None] == t2[None, :], -1e5, d)
    d = jnp.where(ci == cj, d, -1e30)
    m = d.max(axis=-1, keepdims=True)
    e = jnp.exp(d - m)
    s = e.sum(-1, keepdims=True)
    lse = m + jnp.log(s)
    probs = jnp.exp(d - lse)
    so_ref[0] = jnp.dot(probs, v2, preferred_element_type=jnp.float32)
    lg_ref[0, pl.ds(g, 1), :] = lse.reshape(1, GQ)


def _attn(sqk, sv, stf):
    cur = pl.BlockSpec((1, GQ, D_HEAD), lambda b, g: (b, g, 0))
    prev = pl.BlockSpec((1, GQ, D_HEAD), lambda b, g: (b, (g + NG - 1) % NG, 0))
    return pl.pallas_call(
        _attn_body,
        grid=(BH, NG),
        in_specs=[
            cur, prev, cur, prev,
            pl.BlockSpec((1, NG, GQ), lambda b, g: (b, 0, 0)),
        ],
        out_specs=[
            pl.BlockSpec((1, GQ, D_HEAD), lambda b, g: (b, g, 0)),
            pl.BlockSpec((1, NG, GQ), lambda b, g: (b, 0, 0)),
        ],
        out_shape=[
            jax.ShapeDtypeStruct((BH, N_HASHES * L, D_HEAD), jnp.float32),
            jax.ShapeDtypeStruct((BH, NG, GQ), jnp.float32),
        ],
    )(sqk, sqk, sv, sv, stf)


def _combine_body(o4_ref, lg_ref, out_ref):
    lg = lg_ref[0]
    m = lg.max(axis=0, keepdims=True)
    e = jnp.exp(lg - m)
    w = e / e.sum(axis=0, keepdims=True)
    acc = o4_ref[0, 0] * w[0][:, None]
    for h in range(1, N_HASHES):
        acc = acc + o4_ref[0, h] * w[h][:, None]
    out_ref[0] = acc


def _combine(o4, lg):
    return pl.pallas_call(
        _combine_body,
        grid=(BH,),
        in_specs=[
            pl.BlockSpec((1, N_HASHES, L, D_HEAD), lambda b: (b, 0, 0, 0)),
            pl.BlockSpec((1, N_HASHES, L), lambda b: (b, 0, 0)),
        ],
        out_specs=pl.BlockSpec((1, L, D_HEAD), lambda b: (b, 0, 0)),
        out_shape=jax.ShapeDtypeStruct((BH, L, D_HEAD), jnp.float32),
    )(o4, lg)


def _wo_body(a_ref, w_ref, x_ref, out_ref):
    h = pl.program_id(1)
    y = jnp.dot(a_ref[0], w_ref[0], preferred_element_type=jnp.float32)

    @pl.when(h == 0)
    def _():
        out_ref[...] = y

    @pl.when(h > 0)
    def _():
        out_ref[...] += y

    @pl.when(h == N_HEADS - 1)
    def _():
        out_ref[...] = _ln(out_ref[...] + x_ref[...])


def _wo_ln(attn_bh, wo3, x):
    return pl.pallas_call(
        _wo_body,
        grid=(NRB, N_HEADS),
        in_specs=[
            pl.BlockSpec(
                (1, RB, D_HEAD), lambda r, h: ((r // 4) * N_HEADS + h, r % 4, 0)
            ),
            pl.BlockSpec((1, D_HEAD, D_MODEL), lambda r, h: (h, 0, 0)),
            pl.BlockSpec((RB, D_MODEL), lambda r, h: (r, 0)),
        ],
        out_specs=pl.BlockSpec((RB, D_MODEL), lambda r, h: (r, 0)),
        out_shape=jax.ShapeDtypeStruct((B * L, D_MODEL), jnp.float32),
    )(attn_bh, wo3, x)


CC = 512
NCC = D_FF // CC


def _ffn_body(x_ref, w1_ref, w2_ref, out_ref):
    c = pl.program_id(1)
    t = jax.nn.gelu(
        jnp.dot(x_ref[...], w1_ref[...], preferred_element_type=jnp.float32)
    )
    y = jnp.dot(t, w2_ref[...], preferred_element_type=jnp.float32)

    @pl.when(c == 0)
    def _():
        out_ref[...] = y

    @pl.when(c > 0)
    def _():
        out_ref[...] += y

    @pl.when(c == NCC - 1)
    def _():
        out_ref[...] = _ln(out_ref[...] + x_ref[...])


def _ffn_ln(x, w1, w2):
    return pl.pallas_call(
        _ffn_body,
        grid=(NRB, NCC),
        in_specs=[
            pl.BlockSpec((RB, D_MODEL), lambda r, c: (r, 0)),
            pl.BlockSpec((D_MODEL, CC), lambda r, c: (0, c)),
            pl.BlockSpec((CC, D_MODEL), lambda r, c: (c, 0)),
        ],
        out_specs=pl.BlockSpec((RB, D_MODEL), lambda r, c: (r, 0)),
        out_shape=jax.ShapeDtypeStruct((B * L, D_MODEL), jnp.float32),
    )(x, w1, w2)


def _final_body(x_ref, w_ref, out_ref):
    out_ref[...] = jnp.dot(
        _ln(x_ref[...]), w_ref[...], preferred_element_type=jnp.float32
    )


def _final(x, projp):
    return pl.pallas_call(
        _final_body,
        grid=(NRB,),
        in_specs=[
            pl.BlockSpec((RB, D_MODEL), lambda r: (r, 0)),
            pl.BlockSpec((D_MODEL, 128), lambda r: (0, 0)),
        ],
        out_specs=pl.BlockSpec((RB, 128), lambda r: (r, 0)),
        out_shape=jax.ShapeDtypeStruct((B * L, 128), jnp.float32),
    )(x, projp)


def _layer(h, p, rot, buckets_in=None):
    v_bh = _qkv(h, p['wv'])

    qk3 = h.reshape(B, L, D_MODEL) @ p['wqk']
    qk_bh = qk3.reshape(B, L, N_HEADS, D_HEAD).transpose(0, 2, 1, 3)
    qk_bh = qk_bh.reshape(BH, L, D_HEAD)
    if buckets_in is None:
        rotated = jnp.einsum('bld,dhr->bhlr', qk_bh, rot)
        rotated = jnp.concatenate([rotated, -rotated], axis=-1)
        buckets = jnp.argmax(rotated, axis=-1).astype(jnp.int32)
    else:
        buckets = buckets_in

    key = buckets * L + jnp.arange(L, dtype=jnp.int32)[None, None, :]
    st = jnp.argsort(key, axis=-1).astype(jnp.int32)
    undo = jnp.argsort(st, axis=-1).astype(jnp.int32)
    sqk = jnp.take_along_axis(
        qk_bh[:, None], st[..., None], axis=2
    ).reshape(BH, N_HASHES * L, D_HEAD)
    sv = jnp.take_along_axis(
        v_bh[:, None], st[..., None], axis=2
    ).reshape(BH, N_HASHES * L, D_HEAD)
    stf = st.reshape(BH, NG, GQ).astype(jnp.float32)

    so, slog = _attn(sqk, sv, stf)
    slog2 = slog.reshape(BH, N_HASHES * L)
    ug = (undo + (jnp.arange(N_HASHES, dtype=jnp.int32) * L)[None, :, None]
          ).reshape(BH, N_HASHES * L)
    o4 = jnp.take_along_axis(so, ug[:, :, None], axis=1).reshape(
        BH, N_HASHES, L, D_HEAD
    )
    lg = jnp.take_along_axis(slog2, ug, axis=1).reshape(BH, N_HASHES, L)

    attn_bh = _combine(o4, lg)
    h2 = _wo_ln(attn_bh, p['wo'].reshape(N_HEADS, D_HEAD, D_MODEL), h)
    return _ffn_ln(h2, p['w1'], p['w2'])


def kernel(x_enc, x_mark_enc, x_dec, x_mark_dec, params, rotations):
    x = jnp.concatenate([x_enc, x_dec[:, -PRED_LEN:, :]], axis=1)
    xm = jnp.concatenate([x_mark_enc, x_mark_dec[:, -PRED_LEN:, :]], axis=1)
    xp = jnp.pad(x, ((0, 0), (1, 1), (0, 0)), mode='wrap')
    tok = jax.lax.conv_general_dilated(
        xp, params['conv_w'], (1,), 'VALID',
        dimension_numbers=('NWC', 'WIO', 'NWC'))
    h = tok + xm @ params['temp_w'] + jnp.asarray(_pos_emb_np())[None]
    h = h.reshape(B * L, D_MODEL)
    for i in range(E_LAYERS):
        h = _layer(h, params['layers'][i], rotations[i])

    out = _final(h, jnp.pad(params['proj_w'], ((0, 0), (0, 128 - C_OUT))))
    return out[:, :C_OUT].reshape(B, L, C_OUT)[:, -PRED_LEN:, :]


# speedup vs baseline: 1.0536x; 1.0536x over previous
import jax
import jax.numpy as jnp
import numpy as np
from jax.experimental import pallas as pl

B = 2
SEQ_LEN = 1536
PRED_LEN = 512
ENC_IN = 21
C_OUT = 21
D_MODEL = 1024
N_HEADS = 16
D_FF = 2048
E_LAYERS = 2
MARK = 4
BUCKET = 16
N_HASHES = 4
L = SEQ_LEN + PRED_LEN
D_HEAD = D_MODEL // N_HEADS
N_BUCKETS = L // BUCKET
BH = B * N_HEADS
RB = 512
NRB = (B * L) // RB
GQ = 256
NG = (N_HASHES * L) // GQ


def _pos_emb_np():
    pos = np.arange(L)[:, None].astype(np.float64)
    i = np.arange(0, D_MODEL, 2)[None, :].astype(np.float64)
    ang = pos / np.power(10000.0, i / D_MODEL)
    pe = np.zeros((L, D_MODEL), dtype=np.float32)
    pe[:, 0::2] = np.sin(ang)
    pe[:, 1::2] = np.cos(ang)
    return pe


def _ln(x):
    m = x.mean(-1, keepdims=True)
    v = ((x - m) ** 2).mean(-1, keepdims=True)
    return (x - m) / jnp.sqrt(v + 1e-5)


def _qkv_body(x_ref, w_ref, v_ref):
    y = jnp.dot(x_ref[...], w_ref[...], preferred_element_type=jnp.float32)
    for h in range(N_HEADS):
        v_ref[h] = y[:, h * D_HEAD:(h + 1) * D_HEAD]


def _qkv(h, wv):
    out_spec = pl.BlockSpec(
        (N_HEADS, RB, D_HEAD), lambda r: (r // 4, r % 4, 0)
    )
    return pl.pallas_call(
        _qkv_body,
        grid=(NRB,),
        in_specs=[
            pl.BlockSpec((RB, D_MODEL), lambda r: (r, 0)),
            pl.BlockSpec((D_MODEL, D_MODEL), lambda r: (0, 0)),
        ],
        out_specs=out_spec,
        out_shape=jax.ShapeDtypeStruct((BH, L, D_HEAD), jnp.float32),
    )(h, wv)


def _attn_body(qc_ref, qp_ref, vc_ref, vp_ref, t_ref, so_ref, lg_ref):
    g = pl.program_id(1)
    q = qc_ref[0]
    kc = q / (jnp.sqrt((q * q).sum(-1, keepdims=True)) + 1e-9)
    qp = qp_ref[0][GQ - BUCKET:]
    kp = qp / (jnp.sqrt((qp * qp).sum(-1, keepdims=True)) + 1e-9)
    kprev = jnp.concatenate([kp, kc[:GQ - BUCKET]], axis=0)
    k2 = jnp.concatenate([kc, kprev], axis=0)
    vc = vc_ref[0]
    vp = vp_ref[0][GQ - BUCKET:]
    v2 = jnp.concatenate(
        [vc, jnp.concatenate([vp, vc[:GQ - BUCKET]], axis=0)], axis=0
    )
    tcur = t_ref[0, pl.ds(g, 1), :][0]
    tprow = t_ref[0, pl.ds((g + NG - 1) % NG, 1), :][0]
    tprev = jnp.concatenate([tprow[GQ - BUCKET:], tcur[:GQ - BUCKET]])
    t2 = jnp.concatenate([tcur, tprev])

    d = jnp.dot(q, k2.T, preferred_element_type=jnp.float32) * (D_HEAD ** -0.5)
    ci = jax.lax.broadcasted_iota(jnp.int32, (GQ, 2 * GQ), 0) // BUCKET
    cj = (jax.lax.broadcasted_iota(jnp.int32, (GQ, 2 * GQ), 1) % GQ) // BUCKET
    d = jnp.where(tcur[:, None] == t2[None, :], -1e5, d)
    d = jnp.where(ci == cj, d, -1e30)
    m = d.max(axis=-1, keepdims=True)
    e = jnp.exp(d - m)
    s = e.sum(-1, keepdims=True)
    lse = m + jnp.log(s)
    probs = jnp.exp(d - lse)
    so_ref[0] = jnp.dot(probs, v2, preferred_element_type=jnp.float32)
    lg_ref[0, pl.ds(g, 1), :] = lse.reshape(1, GQ)


def _attn(sqk, sv, stf):
    cur = pl.BlockSpec((1, GQ, D_HEAD), lambda b, g: (b, g, 0))
    prev = pl.BlockSpec((1, GQ, D_HEAD), lambda b, g: (b, (g + NG - 1) % NG, 0))
    return pl.pallas_call(
        _attn_body,
        grid=(BH, NG),
        in_specs=[
            cur, prev, cur, prev,
            pl.BlockSpec((1, NG, GQ), lambda b, g: (b, 0, 0)),
        ],
        out_specs=[
            pl.BlockSpec((1, GQ, D_HEAD), lambda b, g: (b, g, 0)),
            pl.BlockSpec((1, NG, GQ), lambda b, g: (b, 0, 0)),
        ],
        out_shape=[
            jax.ShapeDtypeStruct((BH, N_HASHES * L, D_HEAD), jnp.float32),
            jax.ShapeDtypeStruct((BH, NG, GQ), jnp.float32),
        ],
    )(sqk, sqk, sv, sv, stf)


def _combine_body(o4_ref, lg_ref, out_ref):
    lg = lg_ref[0]
    m = lg.max(axis=0, keepdims=True)
    e = jnp.exp(lg - m)
    w = e / e.sum(axis=0, keepdims=True)
    acc = o4_ref[0, 0] * w[0][:, None]
    for h in range(1, N_HASHES):
        acc = acc + o4_ref[0, h] * w[h][:, None]
    out_ref[0] = acc


def _combine(o4, lg):
    return pl.pallas_call(
        _combine_body,
        grid=(BH,),
        in_specs=[
            pl.BlockSpec((1, N_HASHES, L, D_HEAD), lambda b: (b, 0, 0, 0)),
            pl.BlockSpec((1, N_HASHES, L), lambda b: (b, 0, 0)),
        ],
        out_specs=pl.BlockSpec((1, L, D_HEAD), lambda b: (b, 0, 0)),
        out_shape=jax.ShapeDtypeStruct((BH, L, D_HEAD), jnp.float32),
    )(o4, lg)


def _wo_body(a_ref, w_ref, x_ref, out_ref):
    h = pl.program_id(1)
    y = jnp.dot(a_ref[0], w_ref[0], preferred_element_type=jnp.float32)

    @pl.when(h == 0)
    def _():
        out_ref[...] = y

    @pl.when(h > 0)
    def _():
        out_ref[...] += y

    @pl.when(h == N_HEADS - 1)
    def _():
        out_ref[...] = _ln(out_ref[...] + x_ref[...])


def _wo_ln(attn_bh, wo3, x):
    return pl.pallas_call(
        _wo_body,
        grid=(NRB, N_HEADS),
        in_specs=[
            pl.BlockSpec(
                (1, RB, D_HEAD), lambda r, h: ((r // 4) * N_HEADS + h, r % 4, 0)
            ),
            pl.BlockSpec((1, D_HEAD, D_MODEL), lambda r, h: (h, 0, 0)),
            pl.BlockSpec((RB, D_MODEL), lambda r, h: (r, 0)),
        ],
        out_specs=pl.BlockSpec((RB, D_MODEL), lambda r, h: (r, 0)),
        out_shape=jax.ShapeDtypeStruct((B * L, D_MODEL), jnp.float32),
    )(attn_bh, wo3, x)


CC = 512
NCC = D_FF // CC


def _ffn_body(x_ref, w1_ref, w2_ref, out_ref):
    c = pl.program_id(1)
    t = jax.nn.gelu(
        jnp.dot(x_ref[...], w1_ref[...], preferred_element_type=jnp.float32)
    )
    y = jnp.dot(t, w2_ref[...], preferred_element_type=jnp.float32)

    @pl.when(c == 0)
    def _():
        out_ref[...] = y

    @pl.when(c > 0)
    def _():
        out_ref[...] += y

    @pl.when(c == NCC - 1)
    def _():
        out_ref[...] = _ln(out_ref[...] + x_ref[...])


def _ffn_ln(x, w1, w2):
    return pl.pallas_call(
        _ffn_body,
        grid=(NRB, NCC),
        in_specs=[
            pl.BlockSpec((RB, D_MODEL), lambda r, c: (r, 0)),
            pl.BlockSpec((D_MODEL, CC), lambda r, c: (0, c)),
            pl.BlockSpec((CC, D_MODEL), lambda r, c: (c, 0)),
        ],
        out_specs=pl.BlockSpec((RB, D_MODEL), lambda r, c: (r, 0)),
        out_shape=jax.ShapeDtypeStruct((B * L, D_MODEL), jnp.float32),
    )(x, w1, w2)


def _final_body(x_ref, w_ref, out_ref):
    out_ref[...] = jnp.dot(
        _ln(x_ref[...]), w_ref[...], preferred_element_type=jnp.float32
    )


def _final(x, projp):
    return pl.pallas_call(
        _final_body,
        grid=(NRB,),
        in_specs=[
            pl.BlockSpec((RB, D_MODEL), lambda r: (r, 0)),
            pl.BlockSpec((D_MODEL, 128), lambda r: (0, 0)),
        ],
        out_specs=pl.BlockSpec((RB, 128), lambda r: (r, 0)),
        out_shape=jax.ShapeDtypeStruct((B * L, 128), jnp.float32),
    )(x, projp)


def _layer(h, p, rot):
    v_bh = _qkv(h, p['wv'])

    qk3 = h.reshape(B, L, D_MODEL) @ p['wqk']
    qk_bh = qk3.reshape(B, L, N_HEADS, D_HEAD).transpose(0, 2, 1, 3)
    qk_bh = qk_bh.reshape(BH, L, D_HEAD)
    rotated = jnp.einsum('bld,dhr->bhlr', qk_bh, rot)
    rotated = jnp.concatenate([rotated, -rotated], axis=-1)
    buckets = jnp.argmax(rotated, axis=-1).astype(jnp.int32)

    key = buckets * L + jnp.arange(L, dtype=jnp.int32)[None, None, :]
    st = jnp.argsort(key, axis=-1).astype(jnp.int32)
    undo = jnp.argsort(st, axis=-1).astype(jnp.int32)
    sqk = jnp.take_along_axis(
        qk_bh[:, None], st[..., None], axis=2
    ).reshape(BH, N_HASHES * L, D_HEAD)
    sv = jnp.take_along_axis(
        v_bh[:, None], st[..., None], axis=2
    ).reshape(BH, N_HASHES * L, D_HEAD)
    stf = st.reshape(BH, NG, GQ).astype(jnp.float32)

    so, slog = _attn(sqk, sv, stf)
    slog2 = slog.reshape(BH, N_HASHES * L)
    ug = (undo + (jnp.arange(N_HASHES, dtype=jnp.int32) * L)[None, :, None]
          ).reshape(BH, N_HASHES * L)
    o4 = jnp.take_along_axis(so, ug[:, :, None], axis=1).reshape(
        BH, N_HASHES, L, D_HEAD
    )
    lg = jnp.take_along_axis(slog2, ug, axis=1).reshape(BH, N_HASHES, L)

    attn_bh = _combine(o4, lg)
    h2 = _wo_ln(attn_bh, p['wo'].reshape(N_HEADS, D_HEAD, D_MODEL), h)
    return _ffn_ln(h2, p['w1'], p['w2'])


def kernel(x_enc, x_mark_enc, x_dec, x_mark_dec, params, rotations):
    x = jnp.concatenate([x_enc, x_dec[:, -PRED_LEN:, :]], axis=1)
    xm = jnp.concatenate([x_mark_enc, x_mark_dec[:, -PRED_LEN:, :]], axis=1)
    xp = jnp.pad(x, ((0, 0), (1, 1), (0, 0)), mode='wrap')
    tok = jax.lax.conv_general_dilated(
        xp, params['conv_w'], (1,), 'VALID',
        dimension_numbers=('NWC', 'WIO', 'NWC'))
    h = tok + xm @ params['temp_w'] + jnp.asarray(_pos_emb_np())[None]
    h = h.reshape(B * L, D_MODEL)
    for i in range(E_LAYERS):
        h = _layer(h, params['layers'][i], rotations[i])

    out = _final(h, jnp.pad(params['proj_w'], ((0, 0), (0, 128 - C_OUT))))
    return out[:, :C_OUT].reshape(B, L, C_OUT)[:, -PRED_LEN:, :]


# speedup vs baseline: 1.0671x; 1.0128x over previous
import jax
import jax.numpy as jnp
import numpy as np
from jax.experimental import pallas as pl

B = 2
SEQ_LEN = 1536
PRED_LEN = 512
ENC_IN = 21
C_OUT = 21
D_MODEL = 1024
N_HEADS = 16
D_FF = 2048
E_LAYERS = 2
MARK = 4
BUCKET = 16
N_HASHES = 4
L = SEQ_LEN + PRED_LEN
D_HEAD = D_MODEL // N_HEADS
N_BUCKETS = L // BUCKET
BH = B * N_HEADS
RB = 512
NRB = (B * L) // RB
GQ = 512
NG = (N_HASHES * L) // GQ


def _pos_emb_np():
    pos = np.arange(L)[:, None].astype(np.float64)
    i = np.arange(0, D_MODEL, 2)[None, :].astype(np.float64)
    ang = pos / np.power(10000.0, i / D_MODEL)
    pe = np.zeros((L, D_MODEL), dtype=np.float32)
    pe[:, 0::2] = np.sin(ang)
    pe[:, 1::2] = np.cos(ang)
    return pe


def _ln(x):
    m = x.mean(-1, keepdims=True)
    v = ((x - m) ** 2).mean(-1, keepdims=True)
    return (x - m) / jnp.sqrt(v + 1e-5)


def _qkv_body(x_ref, w_ref, v_ref):
    y = jnp.dot(x_ref[...], w_ref[...], preferred_element_type=jnp.float32)
    for h in range(N_HEADS):
        v_ref[h] = y[:, h * D_HEAD:(h + 1) * D_HEAD]


def _qkv(h, wv):
    out_spec = pl.BlockSpec(
        (N_HEADS, RB, D_HEAD), lambda r: (r // 4, r % 4, 0)
    )
    return pl.pallas_call(
        _qkv_body,
        grid=(NRB,),
        in_specs=[
            pl.BlockSpec((RB, D_MODEL), lambda r: (r, 0)),
            pl.BlockSpec((D_MODEL, D_MODEL), lambda r: (0, 0)),
        ],
        out_specs=out_spec,
        out_shape=jax.ShapeDtypeStruct((BH, L, D_HEAD), jnp.float32),
    )(h, wv)


def _attn_body(qc_ref, qp_ref, vc_ref, vp_ref, t_ref, so_ref, lg_ref):
    g = pl.program_id(1)
    q = qc_ref[0]
    kc = q / (jnp.sqrt((q * q).sum(-1, keepdims=True)) + 1e-9)
    qp = qp_ref[0][GQ - BUCKET:]
    kp = qp / (jnp.sqrt((qp * qp).sum(-1, keepdims=True)) + 1e-9)
    kprev = jnp.concatenate([kp, kc[:GQ - BUCKET]], axis=0)
    k2 = jnp.concatenate([kc, kprev], axis=0)
    vc = vc_ref[0]
    vp = vp_ref[0][GQ - BUCKET:]
    v2 = jnp.concatenate(
        [vc, jnp.concatenate([vp, vc[:GQ - BUCKET]], axis=0)], axis=0
    )
    tcur = t_ref[0, pl.ds(g, 1), :][0]
    tprow = t_ref[0, pl.ds((g + NG - 1) % NG, 1), :][0]
    tprev = jnp.concatenate([tprow[GQ - BUCKET:], tcur[:GQ - BUCKET]])
    t2 = jnp.concatenate([tcur, tprev])

    d = jnp.dot(q, k2.T, preferred_element_type=jnp.float32) * (D_HEAD ** -0.5)
    ci = jax.lax.broadcasted_iota(jnp.int32, (GQ, 2 * GQ), 0) // BUCKET
    cj = (jax.lax.broadcasted_iota(jnp.int32, (GQ, 2 * GQ), 1) % GQ) // BUCKET
    d = jnp.where(tcur[:, None] == t2[None, :], -1e5, d)
    d = jnp.where(ci == cj, d, -1e30)
    m = d.max(axis=-1, keepdims=True)
    e = jnp.exp(d - m)
    s = e.sum(-1, keepdims=True)
    lse = m + jnp.log(s)
    probs = jnp.exp(d - lse)
    so_ref[0] = jnp.dot(probs, v2, preferred_element_type=jnp.float32)
    lg_ref[0, pl.ds(g, 1), :] = lse.reshape(1, GQ)


def _attn(sqk, sv, stf):
    cur = pl.BlockSpec((1, GQ, D_HEAD), lambda b, g: (b, g, 0))
    prev = pl.BlockSpec((1, GQ, D_HEAD), lambda b, g: (b, (g + NG - 1) % NG, 0))
    return pl.pallas_call(
        _attn_body,
        grid=(BH, NG),
        in_specs=[
            cur, prev, cur, prev,
            pl.BlockSpec((1, NG, GQ), lambda b, g: (b, 0, 0)),
        ],
        out_specs=[
            pl.BlockSpec((1, GQ, D_HEAD), lambda b, g: (b, g, 0)),
            pl.BlockSpec((1, NG, GQ), lambda b, g: (b, 0, 0)),
        ],
        out_shape=[
            jax.ShapeDtypeStruct((BH, N_HASHES * L, D_HEAD), jnp.float32),
            jax.ShapeDtypeStruct((BH, NG, GQ), jnp.float32),
        ],
    )(sqk, sqk, sv, sv, stf)


def _combine_body(o4_ref, lg_ref, out_ref):
    lg = lg_ref[0]
    m = lg.max(axis=0, keepdims=True)
    e = jnp.exp(lg - m)
    w = e / e.sum(axis=0, keepdims=True)
    acc = o4_ref[0, 0] * w[0][:, None]
    for h in range(1, N_HASHES):
        acc = acc + o4_ref[0, h] * w[h][:, None]
    out_ref[0] = acc


def _combine(o4, lg):
    return pl.pallas_call(
        _combine_body,
        grid=(BH,),
        in_specs=[
            pl.BlockSpec((1, N_HASHES, L, D_HEAD), lambda b: (b, 0, 0, 0)),
            pl.BlockSpec((1, N_HASHES, L), lambda b: (b, 0, 0)),
        ],
        out_specs=pl.BlockSpec((1, L, D_HEAD), lambda b: (b, 0, 0)),
        out_shape=jax.ShapeDtypeStruct((BH, L, D_HEAD), jnp.float32),
    )(o4, lg)


def _wo_body(a_ref, w_ref, x_ref, out_ref):
    h = pl.program_id(1)
    y = jnp.dot(a_ref[0], w_ref[0], preferred_element_type=jnp.float32)

    @pl.when(h == 0)
    def _():
        out_ref[...] = y

    @pl.when(h > 0)
    def _():
        out_ref[...] += y

    @pl.when(h == N_HEADS - 1)
    def _():
        out_ref[...] = _ln(out_ref[...] + x_ref[...])


def _wo_ln(attn_bh, wo3, x):
    return pl.pallas_call(
        _wo_body,
        grid=(NRB, N_HEADS),
        in_specs=[
            pl.BlockSpec(
                (1, RB, D_HEAD), lambda r, h: ((r // 4) * N_HEADS + h, r % 4, 0)
            ),
            pl.BlockSpec((1, D_HEAD, D_MODEL), lambda r, h: (h, 0, 0)),
            pl.BlockSpec((RB, D_MODEL), lambda r, h: (r, 0)),
        ],
        out_specs=pl.BlockSpec((RB, D_MODEL), lambda r, h: (r, 0)),
        out_shape=jax.ShapeDtypeStruct((B * L, D_MODEL), jnp.float32),
    )(attn_bh, wo3, x)


CC = 512
NCC = D_FF // CC


def _ffn_body(x_ref, w1_ref, w2_ref, out_ref):
    c = pl.program_id(1)
    t = jax.nn.gelu(
        jnp.dot(x_ref[...], w1_ref[...], preferred_element_type=jnp.float32)
    )
    y = jnp.dot(t, w2_ref[...], preferred_element_type=jnp.float32)

    @pl.when(c == 0)
    def _():
        out_ref[...] = y

    @pl.when(c > 0)
    def _():
        out_ref[...] += y

    @pl.when(c == NCC - 1)
    def _():
        out_ref[...] = _ln(out_ref[...] + x_ref[...])


def _ffn_ln(x, w1, w2):
    return pl.pallas_call(
        _ffn_body,
        grid=(NRB, NCC),
        in_specs=[
            pl.BlockSpec((RB, D_MODEL), lambda r, c: (r, 0)),
            pl.BlockSpec((D_MODEL, CC), lambda r, c: (0, c)),
            pl.BlockSpec((CC, D_MODEL), lambda r, c: (c, 0)),
        ],
        out_specs=pl.BlockSpec((RB, D_MODEL), lambda r, c: (r, 0)),
        out_shape=jax.ShapeDtypeStruct((B * L, D_MODEL), jnp.float32),
    )(x, w1, w2)


def _final_body(x_ref, w_ref, out_ref):
    out_ref[...] = jnp.dot(
        _ln(x_ref[...]), w_ref[...], preferred_element_type=jnp.float32
    )


def _final(x, projp):
    return pl.pallas_call(
        _final_body,
        grid=(NRB,),
        in_specs=[
            pl.BlockSpec((RB, D_MODEL), lambda r: (r, 0)),
            pl.BlockSpec((D_MODEL, 128), lambda r: (0, 0)),
        ],
        out_specs=pl.BlockSpec((RB, 128), lambda r: (r, 0)),
        out_shape=jax.ShapeDtypeStruct((B * L, 128), jnp.float32),
    )(x, projp)


def _layer(h, p, rot):
    v_bh = _qkv(h, p['wv'])

    qk3 = h.reshape(B, L, D_MODEL) @ p['wqk']
    qk_bh = qk3.reshape(B, L, N_HEADS, D_HEAD).transpose(0, 2, 1, 3)
    qk_bh = qk_bh.reshape(BH, L, D_HEAD)
    rotated = jnp.einsum('bld,dhr->bhlr', qk_bh, rot)
    rotated = jnp.concatenate([rotated, -rotated], axis=-1)
    buckets = jnp.argmax(rotated, axis=-1).astype(jnp.int32)

    key = buckets * L + jnp.arange(L, dtype=jnp.int32)[None, None, :]
    st = jnp.argsort(key, axis=-1).astype(jnp.int32)
    undo = jnp.argsort(st, axis=-1).astype(jnp.int32)
    sqk = jnp.take_along_axis(
        qk_bh[:, None], st[..., None], axis=2
    ).reshape(BH, N_HASHES * L, D_HEAD)
    sv = jnp.take_along_axis(
        v_bh[:, None], st[..., None], axis=2
    ).reshape(BH, N_HASHES * L, D_HEAD)
    stf = st.reshape(BH, NG, GQ).astype(jnp.float32)

    so, slog = _attn(sqk, sv, stf)
    slog2 = slog.reshape(BH, N_HASHES * L)
    ug = (undo + (jnp.arange(N_HASHES, dtype=jnp.int32) * L)[None, :, None]
          ).reshape(BH, N_HASHES * L)
    o4 = jnp.take_along_axis(so, ug[:, :, None], axis=1).reshape(
        BH, N_HASHES, L, D_HEAD
    )
    lg = jnp.take_along_axis(slog2, ug, axis=1).reshape(BH, N_HASHES, L)

    attn_bh = _combine(o4, lg)
    h2 = _wo_ln(attn_bh, p['wo'].reshape(N_HEADS, D_HEAD, D_MODEL), h)
    return _ffn_ln(h2, p['w1'], p['w2'])


def kernel(x_enc, x_mark_enc, x_dec, x_mark_dec, params, rotations):
    x = jnp.concatenate([x_enc, x_dec[:, -PRED_LEN:, :]], axis=1)
    xm = jnp.concatenate([x_mark_enc, x_mark_dec[:, -PRED_LEN:, :]], axis=1)
    xp = jnp.pad(x, ((0, 0), (1, 1), (0, 0)), mode='wrap')
    tok = jax.lax.conv_general_dilated(
        xp, params['conv_w'], (1,), 'VALID',
        dimension_numbers=('NWC', 'WIO', 'NWC'))
    h = tok + xm @ params['temp_w'] + jnp.asarray(_pos_emb_np())[None]
    h = h.reshape(B * L, D_MODEL)
    for i in range(E_LAYERS):
        h = _layer(h, params['layers'][i], rotations[i])

    out = _final(h, jnp.pad(params['proj_w'], ((0, 0), (0, 128 - C_OUT))))
    return out[:, :C_OUT].reshape(B, L, C_OUT)[:, -PRED_LEN:, :]
